# baseline (speedup 1.0000x reference)
import functools

import jax
import jax.numpy as jnp
from jax import lax
from jax.experimental import pallas as pl
from jax.experimental.pallas import tpu as pltpu
from jax.experimental.pallas import tpu_sc as plsc

_B = 16384
_L = 50
_LP = 56
_D = 64
_NC = 2
_NS = 16
_NW = _NC * _NS
_BAGS_W = _B // _NW
_CHUNK = 4
_NCHUNK = _BAGS_W // _CHUNK
_ROWS = _CHUNK * _LP
_NIDX = _BAGS_W * _LP
_WP = 128


def _sc_embedding_bag(idx, w2):
    mesh = plsc.VectorSubcoreMesh(core_axis_name="c", subcore_axis_name="s")

    @functools.partial(
        pl.kernel,
        out_type=jax.ShapeDtypeStruct((_B, _D), jnp.float32),
        mesh=mesh,
        compiler_params=pltpu.CompilerParams(use_tc_tiling_on_sc=False),
        scratch_types=[
            pltpu.VMEM((_NIDX,), jnp.int32),
            pltpu.VMEM((_NIDX,), jnp.int32),
            pltpu.VMEM((2, _ROWS, _WP), jnp.float32),
            pltpu.VMEM((2, _CHUNK, _D), jnp.float32),
            pltpu.SemaphoreType.DMA,
            pltpu.SemaphoreType.DMA,
            pltpu.SemaphoreType.DMA,
            pltpu.SemaphoreType.DMA,
        ],
    )
    def body(idx_hbm, w_hbm, out_hbm, idx_v, pair_v, rows_v, out_v,
             gsem0, gsem1, osem0, osem1):
        gsems = (gsem0, gsem1)
        osems = (osem0, osem1)
        wid = lax.axis_index("s") * _NC + lax.axis_index("c")
        bag0 = wid * _BAGS_W

        pltpu.sync_copy(idx_hbm.at[pl.ds(bag0 * _LP, _NIDX)], idx_v)

        def shift_body(t, carry):
            v = idx_v[pl.ds(t * 16, 16)]
            pair_v[pl.ds(t * 16, 16)] = lax.shift_right_logical(v, 1)
            return carry

        lax.fori_loop(0, _NIDX // 16, shift_body, 0)

        def issue(g, slot):
            for j in range(_CHUNK):
                pltpu.make_async_copy(
                    w_hbm.at[pair_v.at[pl.ds((g * _CHUNK + j) * _LP, _LP)]],
                    rows_v.at[slot, pl.ds(j * _LP, _LP), :],
                    gsems[slot],
                ).start()

        def drain_gather(slot):
            pltpu.make_async_copy(
                w_hbm.at[pl.ds(0, _ROWS), :], rows_v.at[slot], gsems[slot]
            ).wait()

        def drain_out(slot):
            pltpu.make_async_copy(
                out_v.at[slot], out_hbm.at[pl.ds(0, _CHUNK), :], osems[slot]
            ).wait()

        def compute(g, slot):
            def bag_body(i, c2):
                p0 = g * (_CHUNK * _LP) + i * _LP
                r0 = i * _LP
                offs = []
                for c in range((_L + 15) // 16):
                    v = idx_v[pl.ds(p0 + 16 * c, 16)]
                    offs.append((v & 1) * _D)
                o_l = [offs[l // 16][l % 16] for l in range(_L)]
                for d in range(_D // 16):
                    acc = None
                    for l in range(_L):
                        v = rows_v[slot, r0 + l, pl.ds(o_l[l] + d * 16, 16)]
                        acc = v if acc is None else acc + v
                    out_v[slot, i, pl.ds(d * 16, 16)] = (
                        acc * jnp.float32(1.0 / _L)
                    )
                return c2

            lax.fori_loop(0, _CHUNK, bag_body, 0)
            pltpu.make_async_copy(
                out_v.at[slot],
                out_hbm.at[pl.ds(bag0 + g * _CHUNK, _CHUNK), :],
                osems[slot],
            ).start()

        issue(0, 0)

        def pair_body(p, carry):
            for b in range(2):
                g = 2 * p + b

                @pl.when(g + 1 < _NCHUNK)
                def _():
                    issue(g + 1, 1 - b)

                drain_gather(b)

                @pl.when(g >= 2)
                def _():
                    drain_out(b)

                compute(g, b)
            return carry

        lax.fori_loop(0, _NCHUNK // 2, pair_body, 0)
        drain_out(0)
        drain_out(1)

    return body(idx, w2)


def kernel(input, weight):
    idxp = jnp.pad(input.astype(jnp.int32), ((0, 0), (0, _LP - _L)))
    w2 = weight.reshape(weight.shape[0] // 2, 2 * _D)
    return _sc_embedding_bag(idxp.reshape(_B * _LP), w2)


# speedup vs baseline: 5.8023x; 5.8023x over previous
import functools

import jax
import jax.numpy as jnp
from jax import lax
from jax.experimental import pallas as pl
from jax.experimental.pallas import tpu as pltpu
from jax.experimental.pallas import tpu_sc as plsc

_B = 16384
_L = 50
_D = 64
_NC = 2
_NS = 16
_NW = _NC * _NS
_BAGS_W = _B // _NW
_CHUNK = 8
_NCHUNK = _BAGS_W // _CHUNK
_ROWS = _CHUNK * _L
_CPT = 128 // _CHUNK


def _sc_embedding_bag(idx, weight):
    mesh = plsc.VectorSubcoreMesh(core_axis_name="c", subcore_axis_name="s")

    @functools.partial(
        pl.kernel,
        out_type=jax.ShapeDtypeStruct((8, 128, 8, 128), jnp.float32),
        mesh=mesh,
        compiler_params=pltpu.CompilerParams(
            use_tc_tiling_on_sc=False, needs_layout_passes=False
        ),
        scratch_types=[
            pltpu.VMEM((_BAGS_W, _L), jnp.int32),
            pltpu.VMEM((2, _ROWS, _D), jnp.float32),
            pltpu.VMEM((8, 8, 128), jnp.float32),
            pltpu.SemaphoreType.DMA,
            pltpu.SemaphoreType.DMA,
        ],
    )
    def body(idx_hbm, w_hbm, out_hbm, idx_v, rows_v, otile_v, gsem0, gsem1):
        gsems = (gsem0, gsem1)
        wid = lax.axis_index("s") * _NC + lax.axis_index("c")
        bag0 = wid * _BAGS_W
        col0 = wid * (_BAGS_W // 128)

        lanes = lax.iota(jnp.int32, 16)
        r_vecs = [lax.shift_right_logical(lanes + 16 * d, 3) for d in range(4)]
        k_vecs = [(lanes + 16 * d) & 7 for d in range(4)]

        pltpu.sync_copy(idx_hbm.at[pl.ds(bag0, _BAGS_W), :], idx_v)

        def issue(g, slot):
            for j in range(_CHUNK):
                pltpu.make_async_copy(
                    w_hbm.at[idx_v.at[g * _CHUNK + j]],
                    rows_v.at[slot, pl.ds(j * _L, _L), :],
                    gsems[slot],
                ).start()

        def drain_gather(slot):
            pltpu.make_async_copy(
                w_hbm.at[pl.ds(0, _ROWS), :], rows_v.at[slot], gsems[slot]
            ).wait()

        def compute(g, slot):
            def bag_body(i, c2):
                r0 = i * _L
                jl = (g % _CPT) * _CHUNK + i
                j_vec = jnp.full((16,), 0, jnp.int32) + jl
                for d in range(_D // 16):
                    sl = pl.ds(d * 16, 16)
                    acc = rows_v[slot, r0, sl]
                    for l in range(1, _L):
                        acc = acc + rows_v[slot, r0 + l, sl]
                    plsc.store_scatter(
                        otile_v,
                        [r_vecs[d], k_vecs[d], j_vec],
                        acc * jnp.float32(1.0 / _L),
                    )
                return c2

            lax.fori_loop(0, _CHUNK, bag_body, 0)

            @pl.when(g % _CPT == _CPT - 1)
            def _():
                pltpu.sync_copy(otile_v, out_hbm.at[:, col0 + g // _CPT])

        issue(0, 0)

        def pair_body(p, carry):
            for b in range(2):
                g = 2 * p + b

                @pl.when(g + 1 < _NCHUNK)
                def _():
                    issue(g + 1, 1 - b)

                drain_gather(b)
                compute(g, b)
            return carry

        lax.fori_loop(0, _NCHUNK // 2, pair_body, 0)

    return body(idx, weight)


def kernel(input, weight):
    out4 = _sc_embedding_bag(input.astype(jnp.int32), weight)
    return out4.transpose(0, 2, 1, 3).reshape(_D, _B).T
